# baseline (speedup 1.0000x reference)
import functools

import jax
import jax.numpy as jnp
from jax import lax
from jax.experimental import pallas as pl
from jax.experimental.pallas import tpu as pltpu
from jax.experimental.pallas import tpu_sc as plsc

N_NODES = 10000
N_PAD = 10240
GROUPS = 128
GSZ = 80
GPT = GROUPS // 32
TRASH = 8
BATCH = 64
NJ = 12
HID = 64
W = NJ * HID
NCLS = 10
KHOP = 4
CST = 1e-05


def _prologue_body(x_ref, win_ref, bin_ref, wq_ref, bq_ref, wk_ref, bk_ref,
                   wv_ref, bv_ref, q_ref, t_ref, v_ref):
    h = jnp.maximum(x_ref[...] @ win_ref[...] + bin_ref[...], 0.0)
    q = h @ wq_ref[...] + bq_ref[...]
    k = h @ wk_ref[...] + bk_ref[...]
    v = h @ wv_ref[...] + bv_ref[...]
    q = jnp.where(q > 0, q + 1.0, jnp.exp(jnp.minimum(q, 0.0)))
    k = jnp.where(k > 0, k + 1.0, jnp.exp(jnp.minimum(k, 0.0)))
    q_ref[...] = q
    v_ref[...] = v
    cols = [v[:, j:j + 1] * k for j in range(NCLS)]
    cols.append(k)
    cols.append(jnp.zeros_like(k))
    t_ref[...] = jnp.concatenate(cols, axis=1)


def _prologue(x_pad, W_in, b_in, WQ, bQ, WK, bK, WV, bV):
    blk = 1024
    grid = (N_PAD // blk,)
    full = lambda shape: pl.BlockSpec(shape, lambda i: (0,) * len(shape))
    return pl.pallas_call(
        _prologue_body,
        grid=grid,
        in_specs=[
            pl.BlockSpec((blk, 128), lambda i: (i, 0)),
            full((128, HID)), full((HID,)),
            full((HID, HID)), full((HID,)),
            full((HID, HID)), full((HID,)),
            full((HID, NCLS)), full((NCLS,)),
        ],
        out_specs=[
            pl.BlockSpec((blk, HID), lambda i: (i, 0)),
            pl.BlockSpec((blk, W), lambda i: (i, 0)),
            pl.BlockSpec((blk, NCLS), lambda i: (i, 0)),
        ],
        out_shape=[
            jax.ShapeDtypeStruct((N_PAD, HID), jnp.float32),
            jax.ShapeDtypeStruct((N_PAD, W), jnp.float32),
            jax.ShapeDtypeStruct((N_PAD, NCLS), jnp.float32),
        ],
    )(x_pad, W_in, b_in, WQ, bQ, WK, bK, WV, bV)


def _edge_plan(row, col):
    e = row.shape[0]
    max_batches = (e + BATCH - 1) // BATCH + GROUPS
    slots = max_batches * BATCH
    cols, srcs = jax.lax.sort_key_val(col, row)
    off = jnp.searchsorted(cols, jnp.arange(0, N_PAD + 1, GSZ,
                                            dtype=jnp.int32)).astype(jnp.int32)
    cnt = off[1:] - off[:-1]
    nbatch = (cnt + BATCH - 1) // BATCH
    bstart = jnp.concatenate(
        [jnp.zeros((1,), jnp.int32), jnp.cumsum(nbatch)])[:GROUPS]
    g_of_batch = (jnp.searchsorted(bstart, jnp.arange(max_batches,
                                                      dtype=jnp.int32),
                                   side="right") - 1).astype(jnp.int32)
    g_of_batch = jnp.clip(g_of_batch, 0, GROUPS - 1)
    ar = jnp.arange(slots, dtype=jnp.int32)
    g = g_of_batch[ar // BATCH]
    j = ar - (bstart[g] * BATCH - off[g])
    valid = j < off[g + 1]
    jc = jnp.clip(j, 0, e - 1)
    esrc = jnp.where(valid, srcs[jc], (ar * 2711 + 17) % N_NODES)
    edst = jnp.where(valid, cols[jc] - g * GSZ, GSZ + (ar % TRASH))
    meta = jnp.zeros((GROUPS, 16), jnp.int32)
    meta = meta.at[:, 0].set(bstart)
    meta = meta.at[:, 1].set(nbatch)
    return esrc, edst, meta.reshape(GROUPS * 16)


def _hop_body(t_in, esrc, edst, meta, t_out, idx_s, idx_d, rows, meta_v, acc,
              sem):
    c = lax.axis_index("c")
    s = lax.axis_index("s")
    w = c * 16 + s
    zero16 = jnp.zeros((16,), jnp.float32)
    pltpu.sync_copy(meta, meta_v)
    for kk in range(GPT):
        g = w * GPT + kk

        def zr(r, carry):
            for q in range(W // 16):
                acc[r, pl.ds(q * 16, 16)] = zero16
            return carry

        lax.fori_loop(0, GSZ + TRASH, zr, 0)
        mrow = meta_v[pl.ds(g * 16, 16)]
        bst = mrow[0]
        nb = mrow[1]

        def bbody(i, carry):
            gb = bst + i
            pltpu.sync_copy(esrc.at[pl.ds(gb * BATCH, BATCH)], idx_s)
            pltpu.sync_copy(edst.at[pl.ds(gb * BATCH, BATCH)], idx_d)
            pltpu.async_copy(t_in.at[idx_s], rows, sem).wait()

            def ebody(e16, carry2):
                ev = idx_d[pl.ds(e16 * 16, 16)]
                for lane in range(16):
                    d = ev[lane]
                    e = e16 * 16 + lane
                    for q in range(W // 16):
                        plsc.addupdate(acc.at[d, pl.ds(q * 16, 16)],
                                       rows[e, pl.ds(q * 16, 16)])
                return carry2

            lax.fori_loop(0, BATCH // 16, ebody, 0)
            return carry

        lax.fori_loop(0, nb, bbody, 0)
        pltpu.sync_copy(acc.at[pl.ds(0, GSZ)], t_out.at[pl.ds(g * GSZ, GSZ)])


@functools.cache
def _make_hop():
    return pl.kernel(
        _hop_body,
        out_type=jax.ShapeDtypeStruct((N_PAD, W), jnp.float32),
        mesh=plsc.VectorSubcoreMesh(core_axis_name="c", subcore_axis_name="s",
                                    num_cores=2, num_subcores=16),
        scratch_types=[
            pltpu.VMEM((BATCH,), jnp.int32),
            pltpu.VMEM((BATCH,), jnp.int32),
            pltpu.VMEM((BATCH, W), jnp.float32),
            pltpu.VMEM((GROUPS * 16,), jnp.int32),
            pltpu.VMEM((GSZ + TRASH, W), jnp.float32),
            pltpu.SemaphoreType.DMA,
        ],
    )


def _epilogue_body(hw_ref, q_ref, v_ref, t1, t2, t3, t4, out_ref):
    q = q_ref[...]
    hid = v_ref[...] * hw_ref[0]
    for k, tr in enumerate((t1, t2, t3, t4)):
        t2d = tr[...]
        hcols = [jnp.sum(q * t2d[:, j * HID:(j + 1) * HID], axis=1,
                         keepdims=True) for j in range(NCLS + 1)]
        h = jnp.concatenate(hcols[:NCLS], axis=1)
        c = hcols[NCLS] + CST
        hid = hid + hw_ref[k + 1] * (h / c)
    out_ref[...] = hid


def _epilogue(hopwise, q, v, ts):
    blk = 512
    tspec = pl.BlockSpec((blk, W), lambda i: (i, 0))
    return pl.pallas_call(
        _epilogue_body,
        grid=(N_PAD // blk,),
        in_specs=[
            pl.BlockSpec(memory_space=pltpu.SMEM),
            pl.BlockSpec((blk, HID), lambda i: (i, 0)),
            pl.BlockSpec((blk, NCLS), lambda i: (i, 0)),
            tspec, tspec, tspec, tspec,
        ],
        out_specs=pl.BlockSpec((blk, NCLS), lambda i: (i, 0)),
        out_shape=jax.ShapeDtypeStruct((N_NODES, NCLS), jnp.float32),
    )(hopwise, q, v, *ts)


def kernel(x, edge_index, W_in, b_in, WQ, bQ, WK, bK, WV, bV, hopwise, alpha):
    del alpha
    x_pad = jnp.zeros((N_PAD, x.shape[1]), jnp.float32).at[:N_NODES].set(x)
    q, t0, v = _prologue(x_pad, W_in, b_in, WQ, bQ, WK, bK, WV, bV)
    esrc, edst, meta = _edge_plan(edge_index[0], edge_index[1])
    t = t0
    hop = _make_hop()
    ts = []
    for _ in range(KHOP):
        t = hop(t, esrc, edst, meta)
        ts.append(t)
    return _epilogue(hopwise, q, v, ts)
